# baseline (speedup 1.0000x reference)
import dataclasses

import jax
import jax.numpy as jnp
from jax import lax
from jax.experimental import pallas as pl
from jax.experimental.pallas import tpu as pltpu
from jax.experimental.pallas import tpu_sc as plsc

N = 10000
E = 160000
D = 256
HALF = 128

NC = 2
NS = 16
LANES = 16

EP = 163840
EROWS = EP // 128
DEG_ROWS = EROWS // (NC * NS)
AGG_ROWS = EROWS // NS
NA = 10240
NPT = NA // NS
RB = 16
MB = AGG_ROWS // RB

_mesh = plsc.VectorSubcoreMesh(core_axis_name="c", subcore_axis_name="s")

_sc_params = pltpu.CompilerParams()
if "needs_layout_passes" in pltpu.CompilerParams.__dataclass_fields__:
    _sc_params = dataclasses.replace(_sc_params, needs_layout_passes=False)


def _deg_body(dst_hbm, ew_hbm, d0_hbm, d1_hbm,
              dstbuf, ewbuf, degloc, redbuf, outbuf, staging):
    c = lax.axis_index("c")
    s = lax.axis_index("s")
    wid = c * NS + s

    @pl.loop(0, NA, step=LANES)
    def _(i):
        degloc[pl.ds(i, LANES)] = jnp.zeros((LANES,), jnp.float32)

    pltpu.sync_copy(dst_hbm.at[pl.ds(wid * DEG_ROWS, DEG_ROWS)], dstbuf)
    pltpu.sync_copy(ew_hbm.at[pl.ds(wid * DEG_ROWS, DEG_ROWS)], ewbuf)

    @pl.loop(0, DEG_ROWS)
    def _(j):
        for k in range(128 // LANES):
            sl = pl.ds(k * LANES, LANES)
            plsc.addupdate_scatter(degloc, [dstbuf[j, sl]], ewbuf[j, sl])

    pltpu.sync_copy(degloc, staging.at[s])
    plsc.subcore_barrier()

    pltpu.sync_copy(staging.at[pl.ds(0, NS), pl.ds(s * NPT, NPT)], redbuf)

    @pl.loop(0, NPT, step=LANES)
    def _(v):
        sl = pl.ds(v, LANES)
        acc = redbuf[0, sl]
        for r in range(1, NS):
            acc = acc + redbuf[r, sl]
        outbuf[sl] = acc

    @pl.when(c == 0)
    def _():
        pltpu.sync_copy(outbuf, d0_hbm.at[pl.ds(s * NPT, NPT)])

    @pl.when(c == 1)
    def _():
        pltpu.sync_copy(outbuf, d1_hbm.at[pl.ds(s * NPT, NPT)])


_deg_kernel = pl.kernel(
    _deg_body,
    out_type=(
        jax.ShapeDtypeStruct((NA,), jnp.float32),
        jax.ShapeDtypeStruct((NA,), jnp.float32),
    ),
    mesh=_mesh,
    scratch_types=[
        pltpu.VMEM((DEG_ROWS, 128), jnp.int32),
        pltpu.VMEM((DEG_ROWS, 128), jnp.float32),
        pltpu.VMEM((NA,), jnp.float32),
        pltpu.VMEM((NS, NPT), jnp.float32),
        pltpu.VMEM((NPT,), jnp.float32),
        pltpu.VMEM_SHARED((NS, NA), jnp.float32),
    ],
    compiler_params=_sc_params,
)


def _agg_body(h_hbm, src1d_hbm, dst_hbm, ew_hbm, z128_hbm,
              s0_hbm, s1_hbm,
              srcbuf, dstbuf, ewbuf, rows0, rows1, sacc,
              gsem0, gsem1, ssem0, ssem1):
    c = lax.axis_index("c")
    s = lax.axis_index("s")

    pltpu.sync_copy(z128_hbm.at[pl.ds(s * NPT, NPT)], sacc.at[pl.ds(s * NPT, NPT)])
    plsc.subcore_barrier()

    pltpu.sync_copy(src1d_hbm.at[pl.ds(s * AGG_ROWS * 128, AGG_ROWS * 128)],
                    srcbuf)

    coff = c * NA

    @pl.loop(0, AGG_ROWS * 128, step=LANES)
    def _(r):
        sl = pl.ds(r, LANES)
        srcbuf[sl] = srcbuf[sl] + coff

    def gstart(q, buf, sem):
        pltpu.async_copy(h_hbm.at[srcbuf.at[pl.ds(q * 128, 128)]], buf, sem)

    def gwait(q, buf, sem):
        pltpu.make_async_copy(h_hbm.at[srcbuf.at[pl.ds(q * 128, 128)]],
                              buf, sem).wait()

    def scale(lr, buf):
        jf = jnp.full((LANES,), lr, dtype=jnp.int32)

        @pl.loop(0, 128, step=2)
        def _(i):
            for d in range(2):
                i_f = jnp.full((LANES,), i + d, dtype=jnp.int32)
                sc = plsc.load_gather(ewbuf, [jf, i_f])
                for k in range(HALF // LANES):
                    sl = pl.ds(k * LANES, LANES)
                    buf[i + d, sl] = buf[i + d, sl] * sc

    gstart(0, rows0, gsem0)
    gstart(1, rows1, gsem1)

    @pl.loop(0, MB)
    def _(m):
        pltpu.sync_copy(dst_hbm.at[pl.ds(s * AGG_ROWS + m * RB, RB)], dstbuf)
        pltpu.sync_copy(ew_hbm.at[pl.ds(s * AGG_ROWS + m * RB, RB)], ewbuf)

        @pl.loop(0, RB, step=2)
        def _(t):
            q0 = m * RB + t

            gwait(q0, rows0, gsem0)
            scale(t, rows0)
            pltpu.async_copy(rows0, sacc.at[dstbuf.at[t]], ssem0, add=True)

            gwait(q0 + 1, rows1, gsem1)
            scale(t + 1, rows1)
            pltpu.async_copy(rows1, sacc.at[dstbuf.at[t + 1]], ssem1, add=True)

            pltpu.make_async_copy(rows0, sacc.at[dstbuf.at[t]], ssem0).wait()

            @pl.when(q0 + 2 < AGG_ROWS)
            def _():
                gstart(q0 + 2, rows0, gsem0)

            pltpu.make_async_copy(rows1, sacc.at[dstbuf.at[t + 1]], ssem1).wait()

            @pl.when(q0 + 3 < AGG_ROWS)
            def _():
                gstart(q0 + 3, rows1, gsem1)

    plsc.subcore_barrier()

    @pl.when(c == 0)
    def _():
        pltpu.sync_copy(sacc.at[pl.ds(s * NPT, NPT)], s0_hbm.at[pl.ds(s * NPT, NPT)])

    @pl.when(c == 1)
    def _():
        pltpu.sync_copy(sacc.at[pl.ds(s * NPT, NPT)], s1_hbm.at[pl.ds(s * NPT, NPT)])


_agg_kernel = pl.kernel(
    _agg_body,
    out_type=(
        jax.ShapeDtypeStruct((NA, HALF), jnp.float32),
        jax.ShapeDtypeStruct((NA, HALF), jnp.float32),
    ),
    mesh=_mesh,
    scratch_types=[
        pltpu.VMEM((AGG_ROWS * 128,), jnp.int32),
        pltpu.VMEM((RB, 128), jnp.int32),
        pltpu.VMEM((RB, 128), jnp.float32),
        pltpu.VMEM((128, HALF), jnp.float32),
        pltpu.VMEM((128, HALF), jnp.float32),
        pltpu.VMEM_SHARED((NA, HALF), jnp.float32),
        pltpu.SemaphoreType.DMA,
        pltpu.SemaphoreType.DMA,
        pltpu.SemaphoreType.DMA,
        pltpu.SemaphoreType.DMA,
    ],
    compiler_params=_sc_params,
)


def _dis_body(d0_ref, d1_ref, disw_ref):
    disw_ref[...] = lax.rsqrt(d0_ref[...] + d1_ref[...] + 1.0)


def _dis(d0, d1):
    return pl.pallas_call(
        _dis_body,
        out_shape=jax.ShapeDtypeStruct((NA // 128, 128), jnp.float32),
    )(d0.reshape(NA // 128, 128), d1.reshape(NA // 128, 128))


def _mm1_body(x_ref, w1_ref, dis_ref, h_ref):
    h = jnp.dot(x_ref[...], w1_ref[...], preferred_element_type=jnp.float32)
    h_ref[0, :, :] = h * dis_ref[...]


def _mm1(x, W1, disv):
    blk = 1000
    return pl.pallas_call(
        _mm1_body,
        grid=(N // blk, 2),
        in_specs=[
            pl.BlockSpec((blk, D), lambda i, half: (i, 0)),
            pl.BlockSpec((D, HALF), lambda i, half: (0, half)),
            pl.BlockSpec((blk, 1), lambda i, half: (i, 0)),
        ],
        out_specs=pl.BlockSpec((1, blk, HALF), lambda i, half: (half, i, 0)),
        out_shape=jax.ShapeDtypeStruct((2, NA, HALF), jnp.float32),
    )(x, W1, disv)


def _mm2_body(s0_ref, s1_ref, h0_ref, h1_ref, dis_ref, b1_ref, w2_ref, b2_ref,
              out_ref):
    dis = dis_ref[...]
    z0 = (s0_ref[...] + h0_ref[...]) * dis
    z1 = (s1_ref[...] + h1_ref[...]) * dis
    z = jnp.concatenate([z0, z1], axis=1) + b1_ref[...]
    z = jnp.maximum(z, 0.0)
    out_ref[...] = (
        jnp.dot(z, w2_ref[...], preferred_element_type=jnp.float32) + b2_ref[...]
    )


def _mm2(S0, S1, h0, h1, disv, b1, W2, b2):
    blk = 1000
    return pl.pallas_call(
        _mm2_body,
        grid=(N // blk,),
        in_specs=[
            pl.BlockSpec((blk, HALF), lambda i: (i, 0)),
            pl.BlockSpec((blk, HALF), lambda i: (i, 0)),
            pl.BlockSpec((blk, HALF), lambda i: (i, 0)),
            pl.BlockSpec((blk, HALF), lambda i: (i, 0)),
            pl.BlockSpec((blk, 1), lambda i: (i, 0)),
            pl.BlockSpec((1, D), lambda i: (0, 0)),
            pl.BlockSpec((D, D), lambda i: (0, 0)),
            pl.BlockSpec((1, D), lambda i: (0, 0)),
        ],
        out_specs=pl.BlockSpec((blk, D), lambda i: (i, 0)),
        out_shape=jax.ShapeDtypeStruct((N, D), jnp.float32),
    )(S0, S1, h0, h1, disv, b1, W2, b2)


def kernel(x, edge_index, edge_weight, W1, b1, W2, b2):
    pad = EP - E
    src = jnp.concatenate([edge_index[0], jnp.zeros((pad,), jnp.int32)])
    dst = jnp.concatenate([edge_index[1], jnp.zeros((pad,), jnp.int32)])
    ew = jnp.concatenate([edge_weight, jnp.zeros((pad,), jnp.float32)])
    src2d = src.reshape(EROWS, 128)
    dst2d = dst.reshape(EROWS, 128)
    ew2d = ew.reshape(EROWS, 128)

    z128 = jnp.zeros((NA, HALF), jnp.float32)

    d0, d1 = _deg_kernel(dst2d, ew2d)
    disv = _dis(d0, d1).reshape(NA, 1)
    h = _mm1(x, W1, disv)
    hflat = h.reshape(2 * NA, HALF)
    S0, S1 = _agg_kernel(hflat, src, dst2d, ew2d, z128)
    return _mm2(S0[:N], S1[:N], h[0, :N], h[1, :N], disv[:N],
                b1.reshape(1, D), W2, b2.reshape(1, D))


# speedup vs baseline: 1.0478x; 1.0478x over previous
import dataclasses

import jax
import jax.numpy as jnp
from jax import lax
from jax.experimental import pallas as pl
from jax.experimental.pallas import tpu as pltpu
from jax.experimental.pallas import tpu_sc as plsc

N = 10000
E = 160000
D = 256
HALF = 128

NC = 2
NS = 16
LANES = 16

EP = 163840
EROWS = EP // 128
DEG_ROWS = EROWS // (NC * NS)
AGG_ROWS = EROWS // NS
NA = 10240
NPT = NA // NS
RB = 16
MB = AGG_ROWS // RB

_mesh = plsc.VectorSubcoreMesh(core_axis_name="c", subcore_axis_name="s")

_sc_params = pltpu.CompilerParams()
if "needs_layout_passes" in pltpu.CompilerParams.__dataclass_fields__:
    _sc_params = dataclasses.replace(_sc_params, needs_layout_passes=False)


def _deg_body(dst_hbm, ew_hbm, d0_hbm, d1_hbm,
              dstbuf, ewbuf, degloc, redbuf, outbuf, staging):
    c = lax.axis_index("c")
    s = lax.axis_index("s")
    wid = c * NS + s

    @pl.loop(0, NA, step=LANES)
    def _(i):
        degloc[pl.ds(i, LANES)] = jnp.zeros((LANES,), jnp.float32)

    pltpu.sync_copy(dst_hbm.at[pl.ds(wid * DEG_ROWS, DEG_ROWS)], dstbuf)
    pltpu.sync_copy(ew_hbm.at[pl.ds(wid * DEG_ROWS, DEG_ROWS)], ewbuf)

    @pl.loop(0, DEG_ROWS)
    def _(j):
        for k in range(128 // LANES):
            sl = pl.ds(k * LANES, LANES)
            plsc.addupdate_scatter(degloc, [dstbuf[j, sl]], ewbuf[j, sl])

    pltpu.sync_copy(degloc, staging.at[s])
    plsc.subcore_barrier()

    pltpu.sync_copy(staging.at[pl.ds(0, NS), pl.ds(s * NPT, NPT)], redbuf)

    @pl.loop(0, NPT, step=LANES)
    def _(v):
        sl = pl.ds(v, LANES)
        acc = redbuf[0, sl]
        for r in range(1, NS):
            acc = acc + redbuf[r, sl]
        outbuf[sl] = acc

    @pl.when(c == 0)
    def _():
        pltpu.sync_copy(outbuf, d0_hbm.at[pl.ds(s * NPT, NPT)])

    @pl.when(c == 1)
    def _():
        pltpu.sync_copy(outbuf, d1_hbm.at[pl.ds(s * NPT, NPT)])


_deg_kernel = pl.kernel(
    _deg_body,
    out_type=(
        jax.ShapeDtypeStruct((NA,), jnp.float32),
        jax.ShapeDtypeStruct((NA,), jnp.float32),
    ),
    mesh=_mesh,
    scratch_types=[
        pltpu.VMEM((DEG_ROWS, 128), jnp.int32),
        pltpu.VMEM((DEG_ROWS, 128), jnp.float32),
        pltpu.VMEM((NA,), jnp.float32),
        pltpu.VMEM((NS, NPT), jnp.float32),
        pltpu.VMEM((NPT,), jnp.float32),
        pltpu.VMEM_SHARED((NS, NA), jnp.float32),
    ],
    compiler_params=_sc_params,
)


def _agg_body(h_hbm, src1d_hbm, dst_hbm, ew_hbm, z128_hbm,
              s0_hbm, s1_hbm,
              srcbuf, dstbuf, ewbuf, rows0, rows1, sacc, gsem0, gsem1):
    c = lax.axis_index("c")
    s = lax.axis_index("s")

    pltpu.sync_copy(z128_hbm.at[pl.ds(s * NPT, NPT)], sacc.at[pl.ds(s * NPT, NPT)])
    plsc.subcore_barrier()

    pltpu.sync_copy(src1d_hbm.at[pl.ds(s * AGG_ROWS * 128, AGG_ROWS * 128)],
                    srcbuf)

    coff = c * NA

    @pl.loop(0, AGG_ROWS * 128, step=LANES)
    def _(r):
        sl = pl.ds(r, LANES)
        srcbuf[sl] = srcbuf[sl] + coff

    def gstart(q, buf, sem):
        pltpu.async_copy(h_hbm.at[srcbuf.at[pl.ds(q * 128, 128)]], buf, sem)

    def gwait(q, buf, sem):
        pltpu.make_async_copy(h_hbm.at[srcbuf.at[pl.ds(q * 128, 128)]],
                              buf, sem).wait()

    def scale(lr, buf):
        jf = jnp.full((LANES,), lr, dtype=jnp.int32)

        @pl.loop(0, 128, step=2)
        def _(i):
            for d in range(2):
                i_f = jnp.full((LANES,), i + d, dtype=jnp.int32)
                sc = plsc.load_gather(ewbuf, [jf, i_f])
                for k in range(HALF // LANES):
                    sl = pl.ds(k * LANES, LANES)
                    buf[i + d, sl] = buf[i + d, sl] * sc

    gstart(0, rows0, gsem0)
    gstart(1, rows1, gsem1)

    @pl.loop(0, MB)
    def _(m):
        pltpu.sync_copy(dst_hbm.at[pl.ds(s * AGG_ROWS + m * RB, RB)], dstbuf)
        pltpu.sync_copy(ew_hbm.at[pl.ds(s * AGG_ROWS + m * RB, RB)], ewbuf)

        @pl.loop(0, RB, step=2)
        def _(t):
            q0 = m * RB + t

            gwait(q0, rows0, gsem0)
            scale(t, rows0)
            pltpu.sync_copy(rows0, sacc.at[dstbuf.at[t]], add=True)

            @pl.when(q0 + 2 < AGG_ROWS)
            def _():
                gstart(q0 + 2, rows0, gsem0)

            gwait(q0 + 1, rows1, gsem1)
            scale(t + 1, rows1)
            pltpu.sync_copy(rows1, sacc.at[dstbuf.at[t + 1]], add=True)

            @pl.when(q0 + 3 < AGG_ROWS)
            def _():
                gstart(q0 + 3, rows1, gsem1)

    plsc.subcore_barrier()

    @pl.when(c == 0)
    def _():
        pltpu.sync_copy(sacc.at[pl.ds(s * NPT, NPT)], s0_hbm.at[pl.ds(s * NPT, NPT)])

    @pl.when(c == 1)
    def _():
        pltpu.sync_copy(sacc.at[pl.ds(s * NPT, NPT)], s1_hbm.at[pl.ds(s * NPT, NPT)])


_agg_kernel = pl.kernel(
    _agg_body,
    out_type=(
        jax.ShapeDtypeStruct((NA, HALF), jnp.float32),
        jax.ShapeDtypeStruct((NA, HALF), jnp.float32),
    ),
    mesh=_mesh,
    scratch_types=[
        pltpu.VMEM((AGG_ROWS * 128,), jnp.int32),
        pltpu.VMEM((RB, 128), jnp.int32),
        pltpu.VMEM((RB, 128), jnp.float32),
        pltpu.VMEM((128, HALF), jnp.float32),
        pltpu.VMEM((128, HALF), jnp.float32),
        pltpu.VMEM_SHARED((NA, HALF), jnp.float32),
        pltpu.SemaphoreType.DMA,
        pltpu.SemaphoreType.DMA,
    ],
    compiler_params=_sc_params,
)


def _dis_body(d0_ref, d1_ref, disw_ref):
    disw_ref[...] = lax.rsqrt(d0_ref[...] + d1_ref[...] + 1.0)


def _dis(d0, d1):
    return pl.pallas_call(
        _dis_body,
        out_shape=jax.ShapeDtypeStruct((NA // 128, 128), jnp.float32),
    )(d0.reshape(NA // 128, 128), d1.reshape(NA // 128, 128))


def _mm1_body(x_ref, w1_ref, dis_ref, h_ref):
    h = jnp.dot(x_ref[...], w1_ref[...], preferred_element_type=jnp.float32)
    h_ref[0, :, :] = h * dis_ref[...]


def _mm1(x, W1, disv):
    blk = 1000
    return pl.pallas_call(
        _mm1_body,
        grid=(N // blk, 2),
        in_specs=[
            pl.BlockSpec((blk, D), lambda i, half: (i, 0)),
            pl.BlockSpec((D, HALF), lambda i, half: (0, half)),
            pl.BlockSpec((blk, 1), lambda i, half: (i, 0)),
        ],
        out_specs=pl.BlockSpec((1, blk, HALF), lambda i, half: (half, i, 0)),
        out_shape=jax.ShapeDtypeStruct((2, NA, HALF), jnp.float32),
    )(x, W1, disv)


def _mm2_body(s0_ref, s1_ref, h0_ref, h1_ref, dis_ref, b1_ref, w2_ref, b2_ref,
              out_ref):
    dis = dis_ref[...]
    z0 = (s0_ref[...] + h0_ref[...]) * dis
    z1 = (s1_ref[...] + h1_ref[...]) * dis
    z = jnp.concatenate([z0, z1], axis=1) + b1_ref[...]
    z = jnp.maximum(z, 0.0)
    out_ref[...] = (
        jnp.dot(z, w2_ref[...], preferred_element_type=jnp.float32) + b2_ref[...]
    )


def _mm2(S0, S1, h0, h1, disv, b1, W2, b2):
    blk = 1000
    return pl.pallas_call(
        _mm2_body,
        grid=(N // blk,),
        in_specs=[
            pl.BlockSpec((blk, HALF), lambda i: (i, 0)),
            pl.BlockSpec((blk, HALF), lambda i: (i, 0)),
            pl.BlockSpec((blk, HALF), lambda i: (i, 0)),
            pl.BlockSpec((blk, HALF), lambda i: (i, 0)),
            pl.BlockSpec((blk, 1), lambda i: (i, 0)),
            pl.BlockSpec((1, D), lambda i: (0, 0)),
            pl.BlockSpec((D, D), lambda i: (0, 0)),
            pl.BlockSpec((1, D), lambda i: (0, 0)),
        ],
        out_specs=pl.BlockSpec((blk, D), lambda i: (i, 0)),
        out_shape=jax.ShapeDtypeStruct((N, D), jnp.float32),
    )(S0, S1, h0, h1, disv, b1, W2, b2)


def kernel(x, edge_index, edge_weight, W1, b1, W2, b2):
    pad = EP - E
    src = jnp.concatenate([edge_index[0], jnp.zeros((pad,), jnp.int32)])
    dst = jnp.concatenate([edge_index[1], jnp.zeros((pad,), jnp.int32)])
    ew = jnp.concatenate([edge_weight, jnp.zeros((pad,), jnp.float32)])
    src2d = src.reshape(EROWS, 128)
    dst2d = dst.reshape(EROWS, 128)
    ew2d = ew.reshape(EROWS, 128)

    z128 = jnp.zeros((NA, HALF), jnp.float32)

    d0, d1 = _deg_kernel(dst2d, ew2d)
    disv = _dis(d0, d1).reshape(NA, 1)
    h = _mm1(x, W1, disv)
    hflat = h.reshape(2 * NA, HALF)
    S0, S1 = _agg_kernel(hflat, src, dst2d, ew2d, z128)
    return _mm2(S0[:N], S1[:N], h[0, :N], h[1, :N], disv[:N],
                b1.reshape(1, D), W2, b2.reshape(1, D))


# speedup vs baseline: 1.1018x; 1.0515x over previous
import dataclasses

import jax
import jax.numpy as jnp
from jax import lax
from jax.experimental import pallas as pl
from jax.experimental.pallas import tpu as pltpu
from jax.experimental.pallas import tpu_sc as plsc

N = 10000
E = 160000
D = 256
HALF = 128

NC = 2
NS = 16
LANES = 16

EP = 163840
EROWS = EP // 128
DEG_ROWS = EROWS // (NC * NS)
AGG_ROWS = EROWS // NS
NA = 10240
NPT = NA // NS
RB = 16
MB = AGG_ROWS // RB

_mesh = plsc.VectorSubcoreMesh(core_axis_name="c", subcore_axis_name="s")

_sc_params = pltpu.CompilerParams()
if "needs_layout_passes" in pltpu.CompilerParams.__dataclass_fields__:
    _sc_params = dataclasses.replace(_sc_params, needs_layout_passes=False)


def _deg_body(dst_hbm, ew_hbm, d0_hbm, d1_hbm,
              dstbuf, ewbuf, degloc, redbuf, outbuf, staging):
    c = lax.axis_index("c")
    s = lax.axis_index("s")
    wid = c * NS + s

    @pl.loop(0, NA, step=LANES)
    def _(i):
        degloc[pl.ds(i, LANES)] = jnp.zeros((LANES,), jnp.float32)

    pltpu.sync_copy(dst_hbm.at[pl.ds(wid * DEG_ROWS, DEG_ROWS)], dstbuf)
    pltpu.sync_copy(ew_hbm.at[pl.ds(wid * DEG_ROWS, DEG_ROWS)], ewbuf)

    @pl.loop(0, DEG_ROWS)
    def _(j):
        for k in range(128 // LANES):
            sl = pl.ds(k * LANES, LANES)
            plsc.addupdate_scatter(degloc, [dstbuf[j, sl]], ewbuf[j, sl])

    pltpu.sync_copy(degloc, staging.at[s])
    plsc.subcore_barrier()

    pltpu.sync_copy(staging.at[pl.ds(0, NS), pl.ds(s * NPT, NPT)], redbuf)

    @pl.loop(0, NPT, step=LANES)
    def _(v):
        sl = pl.ds(v, LANES)
        acc = redbuf[0, sl]
        for r in range(1, NS):
            acc = acc + redbuf[r, sl]
        outbuf[sl] = acc

    @pl.when(c == 0)
    def _():
        pltpu.sync_copy(outbuf, d0_hbm.at[pl.ds(s * NPT, NPT)])

    @pl.when(c == 1)
    def _():
        pltpu.sync_copy(outbuf, d1_hbm.at[pl.ds(s * NPT, NPT)])


_deg_kernel = pl.kernel(
    _deg_body,
    out_type=(
        jax.ShapeDtypeStruct((NA,), jnp.float32),
        jax.ShapeDtypeStruct((NA,), jnp.float32),
    ),
    mesh=_mesh,
    scratch_types=[
        pltpu.VMEM((DEG_ROWS, 128), jnp.int32),
        pltpu.VMEM((DEG_ROWS, 128), jnp.float32),
        pltpu.VMEM((NA,), jnp.float32),
        pltpu.VMEM((NS, NPT), jnp.float32),
        pltpu.VMEM((NPT,), jnp.float32),
        pltpu.VMEM_SHARED((NS, NA), jnp.float32),
    ],
    compiler_params=_sc_params,
)


def _agg_body(h_hbm, src1d_hbm, dst_hbm, ew_hbm, z128_hbm,
              s0_hbm, s1_hbm,
              srcbuf, dstbuf, ewbuf, rows0, rows1, sacc, gsem0, gsem1):
    c = lax.axis_index("c")
    s = lax.axis_index("s")

    pltpu.sync_copy(z128_hbm.at[pl.ds(s * NPT, NPT)], sacc.at[pl.ds(s * NPT, NPT)])
    plsc.subcore_barrier()

    pltpu.sync_copy(src1d_hbm.at[pl.ds(s * AGG_ROWS * 128, AGG_ROWS * 128)],
                    srcbuf)

    coff = c * NA

    @plsc.parallel_loop(0, AGG_ROWS * 128, step=LANES, unroll=4)
    def _(r):
        sl = pl.ds(r, LANES)
        srcbuf[sl] = srcbuf[sl] + coff

    def gstart(q, buf, sem):
        pltpu.async_copy(h_hbm.at[srcbuf.at[pl.ds(q * 128, 128)]], buf, sem)

    def gwait(q, buf, sem):
        pltpu.make_async_copy(h_hbm.at[srcbuf.at[pl.ds(q * 128, 128)]],
                              buf, sem).wait()

    def scale(lr, buf):
        jf = jnp.full((LANES,), lr, dtype=jnp.int32)

        @plsc.parallel_loop(0, 128, unroll=4)
        def _(i):
            i_f = jnp.full((LANES,), i, dtype=jnp.int32)
            sc = plsc.load_gather(ewbuf, [jf, i_f])
            for k in range(HALF // LANES):
                sl = pl.ds(k * LANES, LANES)
                buf[i, sl] = buf[i, sl] * sc

    gstart(0, rows0, gsem0)
    gstart(1, rows1, gsem1)

    @pl.loop(0, MB)
    def _(m):
        pltpu.sync_copy(dst_hbm.at[pl.ds(s * AGG_ROWS + m * RB, RB)], dstbuf)
        pltpu.sync_copy(ew_hbm.at[pl.ds(s * AGG_ROWS + m * RB, RB)], ewbuf)

        @pl.loop(0, RB, step=2)
        def _(t):
            q0 = m * RB + t

            gwait(q0, rows0, gsem0)
            scale(t, rows0)
            pltpu.sync_copy(rows0, sacc.at[dstbuf.at[t]], add=True)

            @pl.when(q0 + 2 < AGG_ROWS)
            def _():
                gstart(q0 + 2, rows0, gsem0)

            gwait(q0 + 1, rows1, gsem1)
            scale(t + 1, rows1)
            pltpu.sync_copy(rows1, sacc.at[dstbuf.at[t + 1]], add=True)

            @pl.when(q0 + 3 < AGG_ROWS)
            def _():
                gstart(q0 + 3, rows1, gsem1)

    plsc.subcore_barrier()

    @pl.when(c == 0)
    def _():
        pltpu.sync_copy(sacc.at[pl.ds(s * NPT, NPT)], s0_hbm.at[pl.ds(s * NPT, NPT)])

    @pl.when(c == 1)
    def _():
        pltpu.sync_copy(sacc.at[pl.ds(s * NPT, NPT)], s1_hbm.at[pl.ds(s * NPT, NPT)])


_agg_kernel = pl.kernel(
    _agg_body,
    out_type=(
        jax.ShapeDtypeStruct((NA, HALF), jnp.float32),
        jax.ShapeDtypeStruct((NA, HALF), jnp.float32),
    ),
    mesh=_mesh,
    scratch_types=[
        pltpu.VMEM((AGG_ROWS * 128,), jnp.int32),
        pltpu.VMEM((RB, 128), jnp.int32),
        pltpu.VMEM((RB, 128), jnp.float32),
        pltpu.VMEM((128, HALF), jnp.float32),
        pltpu.VMEM((128, HALF), jnp.float32),
        pltpu.VMEM_SHARED((NA, HALF), jnp.float32),
        pltpu.SemaphoreType.DMA,
        pltpu.SemaphoreType.DMA,
    ],
    compiler_params=_sc_params,
)


def _dis_body(d0_ref, d1_ref, disw_ref):
    disw_ref[...] = lax.rsqrt(d0_ref[...] + d1_ref[...] + 1.0)


def _dis(d0, d1):
    return pl.pallas_call(
        _dis_body,
        out_shape=jax.ShapeDtypeStruct((NA // 128, 128), jnp.float32),
    )(d0.reshape(NA // 128, 128), d1.reshape(NA // 128, 128))


def _mm1_body(x_ref, w1_ref, dis_ref, h_ref):
    h = jnp.dot(x_ref[...], w1_ref[...], preferred_element_type=jnp.float32)
    h_ref[0, :, :] = h * dis_ref[...]


def _mm1(x, W1, disv):
    blk = 1000
    return pl.pallas_call(
        _mm1_body,
        grid=(N // blk, 2),
        in_specs=[
            pl.BlockSpec((blk, D), lambda i, half: (i, 0)),
            pl.BlockSpec((D, HALF), lambda i, half: (0, half)),
            pl.BlockSpec((blk, 1), lambda i, half: (i, 0)),
        ],
        out_specs=pl.BlockSpec((1, blk, HALF), lambda i, half: (half, i, 0)),
        out_shape=jax.ShapeDtypeStruct((2, NA, HALF), jnp.float32),
    )(x, W1, disv)


def _mm2_body(s0_ref, s1_ref, h0_ref, h1_ref, dis_ref, b1_ref, w2_ref, b2_ref,
              out_ref):
    dis = dis_ref[...]
    z0 = (s0_ref[...] + h0_ref[...]) * dis
    z1 = (s1_ref[...] + h1_ref[...]) * dis
    z = jnp.concatenate([z0, z1], axis=1) + b1_ref[...]
    z = jnp.maximum(z, 0.0)
    out_ref[...] = (
        jnp.dot(z, w2_ref[...], preferred_element_type=jnp.float32) + b2_ref[...]
    )


def _mm2(S0, S1, h0, h1, disv, b1, W2, b2):
    blk = 1000
    return pl.pallas_call(
        _mm2_body,
        grid=(N // blk,),
        in_specs=[
            pl.BlockSpec((blk, HALF), lambda i: (i, 0)),
            pl.BlockSpec((blk, HALF), lambda i: (i, 0)),
            pl.BlockSpec((blk, HALF), lambda i: (i, 0)),
            pl.BlockSpec((blk, HALF), lambda i: (i, 0)),
            pl.BlockSpec((blk, 1), lambda i: (i, 0)),
            pl.BlockSpec((1, D), lambda i: (0, 0)),
            pl.BlockSpec((D, D), lambda i: (0, 0)),
            pl.BlockSpec((1, D), lambda i: (0, 0)),
        ],
        out_specs=pl.BlockSpec((blk, D), lambda i: (i, 0)),
        out_shape=jax.ShapeDtypeStruct((N, D), jnp.float32),
    )(S0, S1, h0, h1, disv, b1, W2, b2)


def kernel(x, edge_index, edge_weight, W1, b1, W2, b2):
    pad = EP - E
    src = jnp.concatenate([edge_index[0], jnp.zeros((pad,), jnp.int32)])
    dst = jnp.concatenate([edge_index[1], jnp.zeros((pad,), jnp.int32)])
    ew = jnp.concatenate([edge_weight, jnp.zeros((pad,), jnp.float32)])
    src2d = src.reshape(EROWS, 128)
    dst2d = dst.reshape(EROWS, 128)
    ew2d = ew.reshape(EROWS, 128)

    z128 = jnp.zeros((NA, HALF), jnp.float32)

    d0, d1 = _deg_kernel(dst2d, ew2d)
    disv = _dis(d0, d1).reshape(NA, 1)
    h = _mm1(x, W1, disv)
    hflat = h.reshape(2 * NA, HALF)
    S0, S1 = _agg_kernel(hflat, src, dst2d, ew2d, z128)
    return _mm2(S0[:N], S1[:N], h[0, :N], h[1, :N], disv[:N],
                b1.reshape(1, D), W2, b2.reshape(1, D))


# speedup vs baseline: 1.1392x; 1.0340x over previous
import dataclasses

import jax
import jax.numpy as jnp
from jax import lax
from jax.experimental import pallas as pl
from jax.experimental.pallas import tpu as pltpu
from jax.experimental.pallas import tpu_sc as plsc

N = 10000
E = 160000
D = 256
HALF = 128

NC = 2
NS = 16
LANES = 16

EP = 163840
EROWS = EP // 128
DEG_ROWS = EROWS // (NC * NS)
AGG_ROWS = EROWS // NS
NA = 10240
NPT = NA // NS
RB = 16
MB = AGG_ROWS // RB

_mesh = plsc.VectorSubcoreMesh(core_axis_name="c", subcore_axis_name="s")

_sc_params = pltpu.CompilerParams()
if "needs_layout_passes" in pltpu.CompilerParams.__dataclass_fields__:
    _sc_params = dataclasses.replace(_sc_params, needs_layout_passes=False)


def _deg_body(dst_hbm, ew_hbm, d0_hbm, d1_hbm,
              dstbuf, ewbuf, degloc, redbuf, outbuf, staging):
    c = lax.axis_index("c")
    s = lax.axis_index("s")
    wid = c * NS + s

    @pl.loop(0, NA, step=LANES)
    def _(i):
        degloc[pl.ds(i, LANES)] = jnp.zeros((LANES,), jnp.float32)

    pltpu.sync_copy(dst_hbm.at[pl.ds(wid * DEG_ROWS, DEG_ROWS)], dstbuf)
    pltpu.sync_copy(ew_hbm.at[pl.ds(wid * DEG_ROWS, DEG_ROWS)], ewbuf)

    @pl.loop(0, DEG_ROWS)
    def _(j):
        for k in range(128 // LANES):
            sl = pl.ds(k * LANES, LANES)
            plsc.addupdate_scatter(degloc, [dstbuf[j, sl]], ewbuf[j, sl])

    pltpu.sync_copy(degloc, staging.at[s])
    plsc.subcore_barrier()

    pltpu.sync_copy(staging.at[pl.ds(0, NS), pl.ds(s * NPT, NPT)], redbuf)

    @pl.loop(0, NPT, step=LANES)
    def _(v):
        sl = pl.ds(v, LANES)
        acc = redbuf[0, sl]
        for r in range(1, NS):
            acc = acc + redbuf[r, sl]
        outbuf[sl] = acc

    @pl.when(c == 0)
    def _():
        pltpu.sync_copy(outbuf, d0_hbm.at[pl.ds(s * NPT, NPT)])

    @pl.when(c == 1)
    def _():
        pltpu.sync_copy(outbuf, d1_hbm.at[pl.ds(s * NPT, NPT)])


_deg_kernel = pl.kernel(
    _deg_body,
    out_type=(
        jax.ShapeDtypeStruct((NA,), jnp.float32),
        jax.ShapeDtypeStruct((NA,), jnp.float32),
    ),
    mesh=_mesh,
    scratch_types=[
        pltpu.VMEM((DEG_ROWS, 128), jnp.int32),
        pltpu.VMEM((DEG_ROWS, 128), jnp.float32),
        pltpu.VMEM((NA,), jnp.float32),
        pltpu.VMEM((NS, NPT), jnp.float32),
        pltpu.VMEM((NPT,), jnp.float32),
        pltpu.VMEM_SHARED((NS, NA), jnp.float32),
    ],
    compiler_params=_sc_params,
)


def _agg_body(h_hbm, src1d_hbm, dst_hbm, ew_hbm, z128_hbm,
              s0_hbm, s1_hbm,
              srcbuf, dstbuf, ewbuf, rows0, rows1, sacc, gsem0, gsem1):
    c = lax.axis_index("c")
    s = lax.axis_index("s")

    pltpu.sync_copy(z128_hbm.at[pl.ds(s * NPT, NPT)], sacc.at[pl.ds(s * NPT, NPT)])
    plsc.subcore_barrier()

    pltpu.sync_copy(src1d_hbm.at[pl.ds(s * AGG_ROWS * 128, AGG_ROWS * 128)],
                    srcbuf)

    coff = c * NA

    @plsc.parallel_loop(0, AGG_ROWS * 128, step=LANES, unroll=4)
    def _(r):
        sl = pl.ds(r, LANES)
        srcbuf[sl] = srcbuf[sl] + coff

    def gstart(q, buf, sem):
        pltpu.async_copy(h_hbm.at[srcbuf.at[pl.ds(q * 128, 128)]], buf, sem)

    def gwait(q, buf, sem):
        pltpu.make_async_copy(h_hbm.at[srcbuf.at[pl.ds(q * 128, 128)]],
                              buf, sem).wait()

    def scale(lr, buf):
        jf = jnp.full((LANES,), lr, dtype=jnp.int32)

        @plsc.parallel_loop(0, 128, unroll=8)
        def _(i):
            i_f = jnp.full((LANES,), i, dtype=jnp.int32)
            sc = plsc.load_gather(ewbuf, [jf, i_f])
            for k in range(HALF // LANES):
                sl = pl.ds(k * LANES, LANES)
                buf[i, sl] = buf[i, sl] * sc

    gstart(0, rows0, gsem0)
    gstart(1, rows1, gsem1)

    @pl.loop(0, MB)
    def _(m):
        pltpu.sync_copy(dst_hbm.at[pl.ds(s * AGG_ROWS + m * RB, RB)], dstbuf)
        pltpu.sync_copy(ew_hbm.at[pl.ds(s * AGG_ROWS + m * RB, RB)], ewbuf)

        @pl.loop(0, RB, step=2)
        def _(t):
            q0 = m * RB + t

            gwait(q0, rows0, gsem0)
            scale(t, rows0)
            pltpu.sync_copy(rows0, sacc.at[dstbuf.at[t]], add=True)

            @pl.when(q0 + 2 < AGG_ROWS)
            def _():
                gstart(q0 + 2, rows0, gsem0)

            gwait(q0 + 1, rows1, gsem1)
            scale(t + 1, rows1)
            pltpu.sync_copy(rows1, sacc.at[dstbuf.at[t + 1]], add=True)

            @pl.when(q0 + 3 < AGG_ROWS)
            def _():
                gstart(q0 + 3, rows1, gsem1)

    plsc.subcore_barrier()

    @pl.when(c == 0)
    def _():
        pltpu.sync_copy(sacc.at[pl.ds(s * NPT, NPT)], s0_hbm.at[pl.ds(s * NPT, NPT)])

    @pl.when(c == 1)
    def _():
        pltpu.sync_copy(sacc.at[pl.ds(s * NPT, NPT)], s1_hbm.at[pl.ds(s * NPT, NPT)])


_agg_kernel = pl.kernel(
    _agg_body,
    out_type=(
        jax.ShapeDtypeStruct((NA, HALF), jnp.float32),
        jax.ShapeDtypeStruct((NA, HALF), jnp.float32),
    ),
    mesh=_mesh,
    scratch_types=[
        pltpu.VMEM((AGG_ROWS * 128,), jnp.int32),
        pltpu.VMEM((RB, 128), jnp.int32),
        pltpu.VMEM((RB, 128), jnp.float32),
        pltpu.VMEM((128, HALF), jnp.float32),
        pltpu.VMEM((128, HALF), jnp.float32),
        pltpu.VMEM_SHARED((NA, HALF), jnp.float32),
        pltpu.SemaphoreType.DMA,
        pltpu.SemaphoreType.DMA,
    ],
    compiler_params=_sc_params,
)


def _mm1_body(x_ref, w1_ref, d0_ref, d1_ref, h_ref):
    h = jnp.dot(x_ref[...], w1_ref[...], preferred_element_type=jnp.float32)
    dis = lax.rsqrt(d0_ref[...] + d1_ref[...] + 1.0)
    h_ref[0, :, :] = h * dis


def _mm1(x, W1, d0, d1):
    blk = 1000
    return pl.pallas_call(
        _mm1_body,
        grid=(N // blk, 2),
        in_specs=[
            pl.BlockSpec((blk, D), lambda i, half: (i, 0)),
            pl.BlockSpec((D, HALF), lambda i, half: (0, half)),
            pl.BlockSpec((blk, 1), lambda i, half: (i, 0)),
            pl.BlockSpec((blk, 1), lambda i, half: (i, 0)),
        ],
        out_specs=pl.BlockSpec((1, blk, HALF), lambda i, half: (half, i, 0)),
        out_shape=jax.ShapeDtypeStruct((2, NA, HALF), jnp.float32),
    )(x, W1, d0, d1)


def _mm2_body(s0_ref, s1_ref, h0_ref, h1_ref, d0_ref, d1_ref, b1_ref, w2_ref,
              b2_ref, out_ref):
    dis = lax.rsqrt(d0_ref[...] + d1_ref[...] + 1.0)
    z0 = (s0_ref[...] + h0_ref[...]) * dis
    z1 = (s1_ref[...] + h1_ref[...]) * dis
    z = jnp.concatenate([z0, z1], axis=1) + b1_ref[...]
    z = jnp.maximum(z, 0.0)
    out_ref[...] = (
        jnp.dot(z, w2_ref[...], preferred_element_type=jnp.float32) + b2_ref[...]
    )


def _mm2(S0, S1, h0, h1, d0, d1, b1, W2, b2):
    blk = 1000
    return pl.pallas_call(
        _mm2_body,
        grid=(N // blk,),
        in_specs=[
            pl.BlockSpec((blk, HALF), lambda i: (i, 0)),
            pl.BlockSpec((blk, HALF), lambda i: (i, 0)),
            pl.BlockSpec((blk, HALF), lambda i: (i, 0)),
            pl.BlockSpec((blk, HALF), lambda i: (i, 0)),
            pl.BlockSpec((blk, 1), lambda i: (i, 0)),
            pl.BlockSpec((blk, 1), lambda i: (i, 0)),
            pl.BlockSpec((1, D), lambda i: (0, 0)),
            pl.BlockSpec((D, D), lambda i: (0, 0)),
            pl.BlockSpec((1, D), lambda i: (0, 0)),
        ],
        out_specs=pl.BlockSpec((blk, D), lambda i: (i, 0)),
        out_shape=jax.ShapeDtypeStruct((N, D), jnp.float32),
    )(S0, S1, h0, h1, d0, d1, b1, W2, b2)


def kernel(x, edge_index, edge_weight, W1, b1, W2, b2):
    pad = EP - E
    src = jnp.concatenate([edge_index[0], jnp.zeros((pad,), jnp.int32)])
    dst = jnp.concatenate([edge_index[1], jnp.zeros((pad,), jnp.int32)])
    ew = jnp.concatenate([edge_weight, jnp.zeros((pad,), jnp.float32)])
    src2d = src.reshape(EROWS, 128)
    dst2d = dst.reshape(EROWS, 128)
    ew2d = ew.reshape(EROWS, 128)

    z128 = jnp.zeros((NA, HALF), jnp.float32)

    d0, d1 = _deg_kernel(dst2d, ew2d)
    d0c = d0.reshape(NA, 1)
    d1c = d1.reshape(NA, 1)
    h = _mm1(x, W1, d0c, d1c)
    hflat = h.reshape(2 * NA, HALF)
    S0, S1 = _agg_kernel(hflat, src, dst2d, ew2d, z128)
    return _mm2(S0[:N], S1[:N], h[0, :N], h[1, :N], d0c[:N], d1c[:N],
                b1.reshape(1, D), W2, b2.reshape(1, D))
